# baseline (speedup 1.0000x reference)
import jax
import jax.numpy as jnp
from jax.experimental import pallas as pl
from jax.experimental.pallas import tpu as pltpu

TAU_ = 1.0
C_ = 100
CPAD_ = 128
B_ = 2048
D_ = 1024
BLK_ = 256
NB_ = B_ // BLK_

_DN_ROWS = (((0,), (0,)), ((), ()))
_DN_FEAT = (((1,), (1,)), ((), ()))


def _copy(z_hbm, buf, sem, blk):
    return pltpu.make_async_copy(
        z_hbm.at[pl.ds(blk * BLK_, BLK_), :], buf.at[blk], sem.at[blk])


def _loss_kernel(z1_hbm, z2_hbm, lab_ref, labr_ref, out_ref,
                 buf1, buf2, nb1_s, nb2_s, sums_s, sem1, sem2):
    f32 = jnp.float32
    bf16 = jnp.bfloat16
    j = pl.program_id(0)

    @pl.when(j == 0)
    def _prologue():
        for blk in range(NB_):
            _copy(z1_hbm, buf1, sem1, blk).start()
            _copy(z2_hbm, buf2, sem2, blk).start()

    _copy(z1_hbm, buf1, sem1, j).wait()
    _copy(z2_hbm, buf2, sem2, j).wait()

    lab_blk = lab_ref[pl.ds(j * BLK_, BLK_), :]
    col = jax.lax.broadcasted_iota(jnp.int32, (BLK_, CPAD_), 1)
    ohb = (lab_blk == col).astype(bf16)

    def prep(buf, nb_s):
        z = buf[j]
        ss = jnp.sum(z * z, axis=1, keepdims=True)
        inv = jax.lax.rsqrt(jnp.maximum(ss, 1e-24))
        nb = (z * inv).astype(bf16)
        nb_s[pl.ds(j * BLK_, BLK_), :] = nb
        return jax.lax.dot_general(ohb, nb, _DN_ROWS,
                                   preferred_element_type=f32)

    part = prep(buf1, nb1_s) + prep(buf2, nb2_s)

    @pl.when(j == 0)
    def _first():
        sums_s[...] = part

    @pl.when(j > 0)
    def _acc():
        sums_s[...] += part

    @pl.when(j == NB_ - 1)
    def _phase1():
        lab_row = labr_ref[...]
        rowi = jax.lax.broadcasted_iota(jnp.int32, (CPAD_, B_), 0)
        ohT = (lab_row == rowi).astype(f32)
        countsT = 2.0 * jnp.sum(ohT, axis=1, keepdims=True)
        invcT = (1.0 / jnp.maximum(countsT, 1.0)) * (1.0 / TAU_)
        sumsb = sums_s[...].astype(bf16)
        vmaskT = (jax.lax.broadcasted_iota(jnp.int32, (CPAD_, 1), 0)
                  < C_).astype(f32)

        def view_loss(nb_s):
            nb = nb_s[...]
            simT = jax.lax.dot_general(sumsb, nb, _DN_FEAT,
                                       preferred_element_type=f32)
            sim = simT * invcT
            p = jnp.sum(sim * ohT, axis=0, keepdims=True)
            s = jnp.sum(jnp.exp(sim - p) * vmaskT, axis=0, keepdims=True)
            return jnp.log(s) - p

        total = jnp.sum(view_loss(nb1_s) + view_loss(nb2_s),
                        axis=1, keepdims=True)
        out_ref[...] = total * (1.0 / (2.0 * B_))


def kernel(z1_features, z2_features, labels):
    lab2d = labels.astype(jnp.int32).reshape(B_, 1)
    labrow = labels.astype(jnp.int32).reshape(1, B_)
    out = pl.pallas_call(
        _loss_kernel,
        grid=(NB_,),
        in_specs=[
            pl.BlockSpec(memory_space=pltpu.MemorySpace.HBM),
            pl.BlockSpec(memory_space=pltpu.MemorySpace.HBM),
            pl.BlockSpec((B_, 1), lambda j: (0, 0)),
            pl.BlockSpec((1, B_), lambda j: (0, 0)),
        ],
        out_specs=pl.BlockSpec((1, 1), lambda j: (0, 0)),
        out_shape=jax.ShapeDtypeStruct((1, 1), jnp.float32),
        scratch_shapes=[
            pltpu.VMEM((NB_, BLK_, D_), jnp.float32),
            pltpu.VMEM((NB_, BLK_, D_), jnp.float32),
            pltpu.VMEM((B_, D_), jnp.bfloat16),
            pltpu.VMEM((B_, D_), jnp.bfloat16),
            pltpu.VMEM((CPAD_, D_), jnp.float32),
            pltpu.SemaphoreType.DMA((NB_,)),
            pltpu.SemaphoreType.DMA((NB_,)),
        ],
        compiler_params=pltpu.CompilerParams(
            dimension_semantics=("arbitrary",),
            vmem_limit_bytes=100 * 1024 * 1024,
        ),
    )(z1_features, z2_features, lab2d, labrow)
    return out[0, 0]


# speedup vs baseline: 1.0109x; 1.0109x over previous
import jax
import jax.numpy as jnp
from jax.experimental import pallas as pl
from jax.experimental.pallas import tpu as pltpu

TAU_ = 1.0
C_ = 100
CPAD_ = 128
B_ = 2048
D_ = 1024
BLK_ = 512
NB_ = B_ // BLK_

_DN_ROWS = (((0,), (0,)), ((), ()))
_DN_FEAT = (((1,), (1,)), ((), ()))


def _copy(z_hbm, buf, sem, blk):
    return pltpu.make_async_copy(
        z_hbm.at[pl.ds(blk * BLK_, BLK_), :], buf.at[blk], sem.at[blk])


def _loss_kernel(z1_hbm, z2_hbm, lab_ref, labr_ref, out_ref,
                 buf1, buf2, nb1_s, nb2_s, inv1_s, inv2_s, sums_s,
                 sem1, sem2):
    f32 = jnp.float32
    bf16 = jnp.bfloat16
    j = pl.program_id(0)

    @pl.when(j == 0)
    def _prologue():
        for blk in range(NB_):
            _copy(z1_hbm, buf1, sem1, blk).start()
            _copy(z2_hbm, buf2, sem2, blk).start()

    _copy(z1_hbm, buf1, sem1, j).wait()
    _copy(z2_hbm, buf2, sem2, j).wait()

    lab_blk = lab_ref[pl.ds(j * BLK_, BLK_), :]
    col = jax.lax.broadcasted_iota(jnp.int32, (BLK_, CPAD_), 1)
    oh_blk = (lab_blk == col).astype(f32)

    def prep(buf, zb_s, inv_s):
        z = buf[j]
        ss = jnp.sum(z * z, axis=1, keepdims=True)
        inv = jax.lax.rsqrt(jnp.maximum(ss, 1e-24))
        zb = z.astype(bf16)
        zb_s[pl.ds(j * BLK_, BLK_), :] = zb
        inv_s[pl.ds(j * BLK_, BLK_), :] = inv
        ohs = (oh_blk * inv).astype(bf16)
        return jax.lax.dot_general(ohs, zb, _DN_ROWS,
                                   preferred_element_type=f32)

    part = prep(buf1, nb1_s, inv1_s) + prep(buf2, nb2_s, inv2_s)

    @pl.when(j == 0)
    def _first():
        sums_s[...] = part

    @pl.when(j > 0)
    def _acc():
        sums_s[...] += part

    @pl.when(j == NB_ - 1)
    def _phase1():
        lab_row = labr_ref[...]
        rowi = jax.lax.broadcasted_iota(jnp.int32, (CPAD_, B_), 0)
        ohT = (lab_row == rowi).astype(f32)
        countsT = 2.0 * jnp.sum(ohT, axis=1, keepdims=True)
        invcT = (1.0 / jnp.maximum(countsT, 1.0)) * (1.0 / TAU_)
        sumsb = sums_s[...].astype(bf16)
        vmaskT = (jax.lax.broadcasted_iota(jnp.int32, (CPAD_, 1), 0)
                  < C_).astype(f32)

        def view_loss(zb_s, inv_s):
            zb = zb_s[...]
            invT = inv_s[...].reshape(1, B_)
            simT = jax.lax.dot_general(sumsb, zb, _DN_FEAT,
                                       preferred_element_type=f32)
            sim = simT * invcT * invT
            p = jnp.sum(sim * ohT, axis=0, keepdims=True)
            s = jnp.sum(jnp.exp(sim - p) * vmaskT, axis=0, keepdims=True)
            return jnp.log(s) - p

        total = jnp.sum(view_loss(nb1_s, inv1_s) + view_loss(nb2_s, inv2_s),
                        axis=1, keepdims=True)
        out_ref[...] = total * (1.0 / (2.0 * B_))


def kernel(z1_features, z2_features, labels):
    lab2d = labels.astype(jnp.int32).reshape(B_, 1)
    labrow = labels.astype(jnp.int32).reshape(1, B_)
    out = pl.pallas_call(
        _loss_kernel,
        grid=(NB_,),
        in_specs=[
            pl.BlockSpec(memory_space=pltpu.MemorySpace.HBM),
            pl.BlockSpec(memory_space=pltpu.MemorySpace.HBM),
            pl.BlockSpec((B_, 1), lambda j: (0, 0)),
            pl.BlockSpec((1, B_), lambda j: (0, 0)),
        ],
        out_specs=pl.BlockSpec((1, 1), lambda j: (0, 0)),
        out_shape=jax.ShapeDtypeStruct((1, 1), jnp.float32),
        scratch_shapes=[
            pltpu.VMEM((NB_, BLK_, D_), jnp.float32),
            pltpu.VMEM((NB_, BLK_, D_), jnp.float32),
            pltpu.VMEM((B_, D_), jnp.bfloat16),
            pltpu.VMEM((B_, D_), jnp.bfloat16),
            pltpu.VMEM((B_, 1), jnp.float32),
            pltpu.VMEM((B_, 1), jnp.float32),
            pltpu.VMEM((CPAD_, D_), jnp.float32),
            pltpu.SemaphoreType.DMA((NB_,)),
            pltpu.SemaphoreType.DMA((NB_,)),
        ],
        compiler_params=pltpu.CompilerParams(
            dimension_semantics=("arbitrary",),
            vmem_limit_bytes=100 * 1024 * 1024,
        ),
    )(z1_features, z2_features, lab2d, labrow)
    return out[0, 0]


# speedup vs baseline: 1.0290x; 1.0179x over previous
import jax
import jax.numpy as jnp
from jax.experimental import pallas as pl
from jax.experimental.pallas import tpu as pltpu

TAU_ = 1.0
C_ = 100
CPAD_ = 128
B_ = 2048
D_ = 1024
BLK_ = 512
NB_ = B_ // BLK_

_DN_ROWS = (((0,), (0,)), ((), ()))
_DN_FEAT = (((1,), (1,)), ((), ()))


def _copy(z_hbm, buf, sem, blk):
    return pltpu.make_async_copy(
        z_hbm.at[pl.ds(blk * BLK_, BLK_), :], buf.at[blk], sem.at[blk])


def _loss_kernel(z1_hbm, z2_hbm, lab_ref, labr_ref, out_ref,
                 buf1, buf2, nb1_s, nb2_s, sums_s, sem1, sem2):
    f32 = jnp.float32
    bf16 = jnp.bfloat16
    j = pl.program_id(0)

    @pl.when(j == 0)
    def _prologue():
        for blk in range(NB_):
            _copy(z1_hbm, buf1, sem1, blk).start()
            _copy(z2_hbm, buf2, sem2, blk).start()

    _copy(z1_hbm, buf1, sem1, j).wait()
    _copy(z2_hbm, buf2, sem2, j).wait()

    lab_blk = lab_ref[pl.ds(j * BLK_, BLK_), :]
    col = jax.lax.broadcasted_iota(jnp.int32, (BLK_, CPAD_), 1)
    ohb = (lab_blk == col).astype(bf16)

    def prep(buf, nb_s):
        z = buf[j]
        ss = jnp.sum(z * z, axis=1, keepdims=True)
        inv = jax.lax.rsqrt(jnp.maximum(ss, 1e-24))
        nb = (z * inv).astype(bf16)
        nb_s[pl.ds(j * BLK_, BLK_), :] = nb
        return jax.lax.dot_general(ohb, nb, _DN_ROWS,
                                   preferred_element_type=f32)

    part = prep(buf1, nb1_s) + prep(buf2, nb2_s)

    @pl.when(j == 0)
    def _first():
        sums_s[...] = part

    @pl.when(j > 0)
    def _acc():
        sums_s[...] += part

    @pl.when(j == NB_ - 1)
    def _phase1():
        lab_row = labr_ref[...]
        rowi = jax.lax.broadcasted_iota(jnp.int32, (CPAD_, B_), 0)
        ohT = (lab_row == rowi).astype(f32)
        countsT = 2.0 * jnp.sum(ohT, axis=1, keepdims=True)
        invcT = (1.0 / jnp.maximum(countsT, 1.0)) * (1.0 / TAU_)
        sumsb = sums_s[...].astype(bf16)
        vmaskT = (jax.lax.broadcasted_iota(jnp.int32, (CPAD_, 1), 0)
                  < C_).astype(f32)

        def view_loss(nb_s):
            nb = nb_s[...]
            simT = jax.lax.dot_general(sumsb, nb, _DN_FEAT,
                                       preferred_element_type=f32)
            sim = simT * invcT
            p = jnp.sum(sim * ohT, axis=0, keepdims=True)
            s = jnp.sum(jnp.exp(sim) * vmaskT, axis=0, keepdims=True)
            return jnp.log(s) - 2.0 * p

        total = jnp.sum(view_loss(nb1_s) + view_loss(nb2_s),
                        axis=1, keepdims=True)
        out_ref[...] = total * (1.0 / (2.0 * B_))


def kernel(z1_features, z2_features, labels):
    lab2d = labels.astype(jnp.int32).reshape(B_, 1)
    labrow = labels.astype(jnp.int32).reshape(1, B_)
    out = pl.pallas_call(
        _loss_kernel,
        grid=(NB_,),
        in_specs=[
            pl.BlockSpec(memory_space=pltpu.MemorySpace.HBM),
            pl.BlockSpec(memory_space=pltpu.MemorySpace.HBM),
            pl.BlockSpec((B_, 1), lambda j: (0, 0)),
            pl.BlockSpec((1, B_), lambda j: (0, 0)),
        ],
        out_specs=pl.BlockSpec((1, 1), lambda j: (0, 0)),
        out_shape=jax.ShapeDtypeStruct((1, 1), jnp.float32),
        scratch_shapes=[
            pltpu.VMEM((NB_, BLK_, D_), jnp.float32),
            pltpu.VMEM((NB_, BLK_, D_), jnp.float32),
            pltpu.VMEM((B_, D_), jnp.bfloat16),
            pltpu.VMEM((B_, D_), jnp.bfloat16),
            pltpu.VMEM((CPAD_, D_), jnp.float32),
            pltpu.SemaphoreType.DMA((NB_,)),
            pltpu.SemaphoreType.DMA((NB_,)),
        ],
        compiler_params=pltpu.CompilerParams(
            dimension_semantics=("arbitrary",),
            vmem_limit_bytes=100 * 1024 * 1024,
        ),
    )(z1_features, z2_features, lab2d, labrow)
    return out[0, 0]
